# baseline (speedup 1.0000x reference)
import functools

import jax
import jax.numpy as jnp
from jax import lax
from jax.experimental import pallas as pl
from jax.experimental.pallas import tpu as pltpu
from jax.experimental.pallas import tpu_sc as plsc

N = 10000
E = 320000
D = 128
EPS = 1e-5

NC = 2
NS = 16
NW = NC * NS

K = 96
WPW = 105
EPW = WPW * K
EP = EPW * NW
ZF = 640
ZL = N - (NS - 1) * ZF


NBUF = 3
PFD = 2


def _sc_scatter_body(h_hbm, src1d, dst1d, ew1d, zeros_hbm, part,
                     src_l, d0, d1, d2, e0, e1, e2, r0, r1, r2, acc,
                     gs0, gs1, gs2, ss0, ss1, ss2, ds0, ds1, ds2,
                     es0, es1, es2):
    dst_w = [d0, d1, d2]
    ew_w = [e0, e1, e2]
    rows = [r0, r1, r2]
    gsem = [gs0, gs1, gs2]
    ssem = [ss0, ss1, ss2]
    dsem = [ds0, ds1, ds2]
    esem = [es0, es1, es2]

    c = lax.axis_index("c")
    s = lax.axis_index("s")
    w = c * NS + s

    pltpu.sync_copy(src1d.at[pl.ds(w * EPW, EPW)], src_l)

    @pl.when(s < NS - 1)
    def _():
        pltpu.sync_copy(zeros_hbm, acc.at[pl.ds(s * ZF, ZF)])

    @pl.when(s == NS - 1)
    def _():
        pltpu.sync_copy(zeros_hbm.at[pl.ds(0, ZL)], acc.at[pl.ds(s * ZF, ZL)])

    plsc.subcore_barrier()

    zi = jnp.zeros((16,), jnp.int32)
    dnums = lax.GatherDimensionNumbers(offset_dims=(),
                                       collapsed_slice_dims=(0,),
                                       start_index_map=(0,))

    def fetch_start(i, b):
        pltpu.async_copy(dst1d.at[pl.ds(w * EPW + i * K, K)], dst_w[b],
                         dsem[b])
        pltpu.async_copy(ew1d.at[pl.ds(w * EPW + i * K, K)], ew_w[b],
                         esem[b])
        pltpu.async_copy(h_hbm.at[src_l.at[pl.ds(i * K, K)]], rows[b],
                         gsem[b])

    def gather_wait(b):
        pltpu.make_async_copy(h_hbm.at[src_l.at[pl.ds(0, K)]], rows[b],
                              gsem[b]).wait()

    def scatter_wait(b):
        pltpu.make_async_copy(rows[b], acc.at[dst_w[b]],
                              ssem[b]).wait()

    def compute(b):
        pltpu.make_async_copy(h_hbm.at[src_l.at[pl.ds(0, K)]], rows[b],
                              gsem[b]).wait()
        pltpu.make_async_copy(dst1d.at[pl.ds(0, K)], dst_w[b],
                              dsem[b]).wait()
        pltpu.make_async_copy(ew1d.at[pl.ds(0, K)], ew_w[b],
                              esem[b]).wait()
        for g16 in range(K // 16):
            chunk = ew_w[b][pl.ds(g16 * 16, 16)]
            for lane in range(16):
                ewb = lax.gather(
                    chunk, (zi + lane)[:, None], dnums, slice_sizes=(1,),
                    mode=lax.GatherScatterMode.PROMISE_IN_BOUNDS)
                e = g16 * 16 + lane
                for f in range(D // 16):
                    sl = pl.ds(f * 16, 16)
                    rows[b][e, sl] = rows[b][e, sl] * ewb
        pltpu.async_copy(rows[b], acc.at[dst_w[b]], ssem[b], add=True)

    for b in range(PFD):
        fetch_start(b, b)

    def trip(t, carry):
        for b in range(NBUF):
            i = t * NBUF + b
            bj = (b + PFD) % NBUF
            compute(b)

            if b == 0:
                @pl.when(t > 0)
                def _():
                    scatter_wait(bj)

                fetch_start(i + PFD, bj)
            else:
                @pl.when(i + PFD < WPW)
                def _(i=i, bj=bj):
                    scatter_wait(bj)
                    fetch_start(i + PFD, bj)

        return carry

    lax.fori_loop(0, WPW // NBUF, trip, 0)

    for b in range(NBUF):
        scatter_wait(b)
    plsc.subcore_barrier()

    @pl.when(s < NS - 1)
    def _():
        pltpu.sync_copy(acc.at[pl.ds(s * ZF, ZF)],
                        part.at[c, pl.ds(s * ZF, ZF)])

    @pl.when(s == NS - 1)
    def _():
        pltpu.sync_copy(acc.at[pl.ds(s * ZF, ZL)],
                        part.at[c, pl.ds(s * ZF, ZL)])


_sc_scatter = functools.partial(
    pl.kernel,
    out_type=jax.ShapeDtypeStruct((NC, N, D), jnp.float32),
    mesh=plsc.VectorSubcoreMesh(core_axis_name="c", subcore_axis_name="s"),
    scratch_types=(
        [pltpu.VMEM((EPW,), jnp.int32)]
        + [pltpu.VMEM((K,), jnp.int32) for _ in range(NBUF)]
        + [pltpu.VMEM((K,), jnp.float32) for _ in range(NBUF)]
        + [pltpu.VMEM((K, D), jnp.float32) for _ in range(NBUF)]
        + [pltpu.VMEM_SHARED((N, D), jnp.float32)]
        + [pltpu.SemaphoreType.DMA for _ in range(4 * NBUF)]
    ),
)(_sc_scatter_body)


def _mm_kernel(x_ref, w_ref, o_ref):
    o_ref[...] = jnp.dot(x_ref[...], w_ref[...],
                         preferred_element_type=jnp.float32)


def _act_mm_kernel(p_ref, b_ref, s_ref, t_ref, w_ref, o_ref):
    m = p_ref[0] + p_ref[1] + b_ref[...]
    a = jnp.maximum(m * s_ref[...] + t_ref[...], 0.0)
    o_ref[...] = jnp.dot(a, w_ref[...], preferred_element_type=jnp.float32)


def _act_kernel(p_ref, b_ref, s_ref, t_ref, o_ref):
    m = p_ref[0] + p_ref[1] + b_ref[...]
    o_ref[...] = jnp.maximum(m * s_ref[...] + t_ref[...], 0.0)


_MB = 1000
_GRID = (N // _MB,)


def _matmul(x, W):
    return pl.pallas_call(
        _mm_kernel,
        grid=_GRID,
        in_specs=[pl.BlockSpec((_MB, D), lambda i: (i, 0)),
                  pl.BlockSpec((D, D), lambda i: (0, 0))],
        out_specs=pl.BlockSpec((_MB, D), lambda i: (i, 0)),
        out_shape=jax.ShapeDtypeStruct((N, D), jnp.float32),
    )(x, W)


def _act_matmul(part, b, scale, beta, W):
    vec = pl.BlockSpec((1, D), lambda i: (0, 0))
    return pl.pallas_call(
        _act_mm_kernel,
        grid=_GRID,
        in_specs=[pl.BlockSpec((NC, _MB, D), lambda i: (0, i, 0)),
                  vec, vec, vec,
                  pl.BlockSpec((D, D), lambda i: (0, 0))],
        out_specs=pl.BlockSpec((_MB, D), lambda i: (i, 0)),
        out_shape=jax.ShapeDtypeStruct((N, D), jnp.float32),
    )(part, b, scale, beta, W)


def _act_only(part, b, scale, beta):
    vec = pl.BlockSpec((1, D), lambda i: (0, 0))
    return pl.pallas_call(
        _act_kernel,
        grid=_GRID,
        in_specs=[pl.BlockSpec((NC, _MB, D), lambda i: (0, i, 0)),
                  vec, vec, vec],
        out_specs=pl.BlockSpec((_MB, D), lambda i: (i, 0)),
        out_shape=jax.ShapeDtypeStruct((N, D), jnp.float32),
    )(part, b, scale, beta)


def kernel(x, edge_index, edge_weight, W1, b1, W2, b2,
           gamma1, beta1, gamma2, beta2):
    pad = EP - E
    src = jnp.concatenate([edge_index[0].astype(jnp.int32),
                           (jnp.arange(pad, dtype=jnp.int32) * 97) % N])
    dst = jnp.concatenate([edge_index[1].astype(jnp.int32),
                           (jnp.arange(pad, dtype=jnp.int32) * 37) % N])
    eww = jnp.concatenate([edge_weight.astype(jnp.float32),
                           jnp.zeros((pad,), jnp.float32)])
    zeros = jnp.zeros((ZF, D), jnp.float32)

    inv = 1.0 / jnp.sqrt(jnp.float32(1.0) + EPS)
    s1 = (gamma1 * inv).reshape(1, D)
    s2 = (gamma2 * inv).reshape(1, D)
    b1r, t1 = b1.reshape(1, D), beta1.reshape(1, D)
    b2r, t2 = b2.reshape(1, D), beta2.reshape(1, D)

    h1 = _matmul(x, W1)
    p1 = _sc_scatter(h1, src, dst, eww, zeros)
    h2 = _act_matmul(p1, b1r, s1, t1, W2)
    p2 = _sc_scatter(h2, src, dst, eww, zeros)
    return _act_only(p2, b2r, s2, t2)


# speedup vs baseline: 1.0564x; 1.0564x over previous
import functools

import numpy as np

import jax
import jax.numpy as jnp
from jax import lax
from jax.experimental import pallas as pl
from jax.experimental.pallas import tpu as pltpu
from jax.experimental.pallas import tpu_sc as plsc

N = 10000
E = 320000
D = 128
EPS = 1e-5

NC = 2
NS = 16
NW = NC * NS

K = 48
WPW = 210
EPW = WPW * K
EP = EPW * NW
ZF = 640
ZL = N - (NS - 1) * ZF


NBUF = 3
PFD = 2


def _sc_scatter_body(h_hbm, src1d, dst1d, ew1d, zeros_hbm, part,
                     src_l, d0, d1, d2, e0, e1, e2, r0, r1, r2,
                     m0, m1, m2, acc,
                     gs0, gs1, gs2, ss0, ss1, ss2, ds0, ds1, ds2,
                     es0, es1, es2):
    dst_w = [d0, d1, d2]
    ew_w = [e0, e1, e2]
    rows = [r0, r1, r2]
    msg = [m0, m1, m2]
    gsem = [gs0, gs1, gs2]
    ssem = [ss0, ss1, ss2]
    dsem = [ds0, ds1, ds2]
    esem = [es0, es1, es2]

    c = lax.axis_index("c")
    s = lax.axis_index("s")
    w = c * NS + s

    pltpu.sync_copy(src1d.at[pl.ds(w * EPW, EPW)], src_l)

    @pl.when(s < NS - 1)
    def _():
        pltpu.sync_copy(zeros_hbm, acc.at[pl.ds(s * ZF, ZF)])

    @pl.when(s == NS - 1)
    def _():
        pltpu.sync_copy(zeros_hbm.at[pl.ds(0, ZL)], acc.at[pl.ds(s * ZF, ZL)])

    plsc.subcore_barrier()

    zi = jnp.zeros((16,), jnp.int32)
    dnums = lax.GatherDimensionNumbers(offset_dims=(),
                                       collapsed_slice_dims=(0,),
                                       start_index_map=(0,))

    def fetch_start(i, b):
        pltpu.async_copy(dst1d.at[pl.ds(w * EPW + i * K, K)], dst_w[b],
                         dsem[b])
        pltpu.async_copy(ew1d.at[pl.ds(w * EPW + i * K, K)], ew_w[b],
                         esem[b])
        pltpu.async_copy(h_hbm.at[src_l.at[pl.ds(i * K, K)]], rows[b],
                         gsem[b])

    def gather_wait(b):
        pltpu.make_async_copy(h_hbm.at[src_l.at[pl.ds(0, K)]], rows[b],
                              gsem[b]).wait()

    def scatter_wait(b):
        pltpu.make_async_copy(msg[b], acc.at[dst_w[b]],
                              ssem[b]).wait()

    def compute(b):
        pltpu.make_async_copy(h_hbm.at[src_l.at[pl.ds(0, K)]], rows[b],
                              gsem[b]).wait()
        pltpu.make_async_copy(dst1d.at[pl.ds(0, K)], dst_w[b],
                              dsem[b]).wait()
        pltpu.make_async_copy(ew1d.at[pl.ds(0, K)], ew_w[b],
                              esem[b]).wait()
        for g16 in range(K // 16):
            chunk = ew_w[b][pl.ds(g16 * 16, 16)]
            for lane in range(16):
                ewb = lax.gather(
                    chunk, (zi + lane)[:, None], dnums, slice_sizes=(1,),
                    mode=lax.GatherScatterMode.PROMISE_IN_BOUNDS)
                e = g16 * 16 + lane
                for c4 in range(D // 32):
                    abi = rows[b][e, pl.ds(c4 * 16, 16)]
                    ab = plsc.bitcast(abi, jnp.bfloat16)
                    fa, fb = plsc.unpack(ab, format=plsc.PackFormat.INTERLEAVED)
                    msg[b][e, pl.ds(c4 * 32, 16)] = fa * ewb
                    msg[b][e, pl.ds(c4 * 32 + 16, 16)] = fb * ewb
        pltpu.async_copy(msg[b], acc.at[dst_w[b]], ssem[b], add=True)

    for b in range(PFD):
        fetch_start(b, b)

    def trip(t, carry):
        for b in range(NBUF):
            i = t * NBUF + b
            bj = (b + PFD) % NBUF
            compute(b)

            if b == 0:
                @pl.when(t > 0)
                def _():
                    scatter_wait(bj)

                fetch_start(i + PFD, bj)
            else:
                @pl.when(i + PFD < WPW)
                def _(i=i, bj=bj):
                    scatter_wait(bj)
                    fetch_start(i + PFD, bj)

        return carry

    lax.fori_loop(0, WPW // NBUF, trip, 0)

    for b in range(NBUF):
        scatter_wait(b)
    plsc.subcore_barrier()

    @pl.when(s < NS - 1)
    def _():
        pltpu.sync_copy(acc.at[pl.ds(s * ZF, ZF)],
                        part.at[c, pl.ds(s * ZF, ZF)])

    @pl.when(s == NS - 1)
    def _():
        pltpu.sync_copy(acc.at[pl.ds(s * ZF, ZL)],
                        part.at[c, pl.ds(s * ZF, ZL)])


_sc_scatter = functools.partial(
    pl.kernel,
    out_type=jax.ShapeDtypeStruct((NC, N, D), jnp.float32),
    mesh=plsc.VectorSubcoreMesh(core_axis_name="c", subcore_axis_name="s"),
    compiler_params=pltpu.CompilerParams(needs_layout_passes=False,
                                         use_tc_tiling_on_sc=False),
    scratch_types=(
        [pltpu.VMEM((EPW,), jnp.int32)]
        + [pltpu.VMEM((K,), jnp.int32) for _ in range(NBUF)]
        + [pltpu.VMEM((K,), jnp.float32) for _ in range(NBUF)]
        + [pltpu.VMEM((K, D // 2), jnp.int32) for _ in range(NBUF)]
        + [pltpu.VMEM((K, D), jnp.float32) for _ in range(NBUF)]
        + [pltpu.VMEM_SHARED((N, D), jnp.float32)]
        + [pltpu.SemaphoreType.DMA for _ in range(4 * NBUF)]
    ),
)(_sc_scatter_body)


def _mm_kernel(x_ref, w_ref, o_ref):
    o_ref[...] = jnp.dot(x_ref[...], w_ref[...],
                         preferred_element_type=jnp.float32
                         ).astype(jnp.bfloat16)


def _act_mm_kernel(p_ref, b_ref, s_ref, t_ref, w_ref, o_ref):
    m = p_ref[0] + p_ref[1] + b_ref[...]
    a = jnp.maximum(m * s_ref[...] + t_ref[...], 0.0)
    o_ref[...] = jnp.dot(a, w_ref[...], preferred_element_type=jnp.float32
                         ).astype(jnp.bfloat16)


def _act_kernel(p_ref, b_ref, s_ref, t_ref, o_ref):
    m = p_ref[0] + p_ref[1] + b_ref[...]
    o_ref[...] = jnp.maximum(m * s_ref[...] + t_ref[...], 0.0)


_MB = 1000
_GRID = (N // _MB,)


def _matmul(x, W):
    return pl.pallas_call(
        _mm_kernel,
        grid=_GRID,
        in_specs=[pl.BlockSpec((_MB, D), lambda i: (i, 0)),
                  pl.BlockSpec((D, D), lambda i: (0, 0))],
        out_specs=pl.BlockSpec((_MB, D), lambda i: (i, 0)),
        out_shape=jax.ShapeDtypeStruct((N, D), jnp.bfloat16),
    )(x, W)


def _act_matmul(part, b, scale, beta, W):
    vec = pl.BlockSpec((1, D), lambda i: (0, 0))
    return pl.pallas_call(
        _act_mm_kernel,
        grid=_GRID,
        in_specs=[pl.BlockSpec((NC, _MB, D), lambda i: (0, i, 0)),
                  vec, vec, vec,
                  pl.BlockSpec((D, D), lambda i: (0, 0))],
        out_specs=pl.BlockSpec((_MB, D), lambda i: (i, 0)),
        out_shape=jax.ShapeDtypeStruct((N, D), jnp.bfloat16),
    )(part, b, scale, beta, W)


def _act_only(part, b, scale, beta):
    vec = pl.BlockSpec((1, D), lambda i: (0, 0))
    return pl.pallas_call(
        _act_kernel,
        grid=_GRID,
        in_specs=[pl.BlockSpec((NC, _MB, D), lambda i: (0, i, 0)),
                  vec, vec, vec],
        out_specs=pl.BlockSpec((_MB, D), lambda i: (i, 0)),
        out_shape=jax.ShapeDtypeStruct((N, D), jnp.float32),
    )(part, b, scale, beta)


def _pack32(h):
    return lax.bitcast_convert_type(h.reshape(N, D // 2, 2), jnp.int32)


def kernel(x, edge_index, edge_weight, W1, b1, W2, b2,
           gamma1, beta1, gamma2, beta2):
    pad = EP - E
    src = jnp.concatenate([edge_index[0].astype(jnp.int32),
                           (jnp.arange(pad, dtype=jnp.int32) * 97) % N])
    dst = jnp.concatenate([edge_index[1].astype(jnp.int32),
                           (jnp.arange(pad, dtype=jnp.int32) * 37) % N])
    eww = jnp.concatenate([edge_weight.astype(jnp.float32),
                           jnp.zeros((pad,), jnp.float32)])
    zeros = jnp.zeros((ZF, D), jnp.float32)

    origin = np.empty((D,), np.int32)
    for mblk in range(D // 32):
        for kk in range(16):
            origin[32 * mblk + 2 * kk] = 32 * mblk + kk
            origin[32 * mblk + 2 * kk + 1] = 32 * mblk + 16 + kk
    W1p = W1[:, origin]
    W2p = W2[:, origin]

    inv = 1.0 / jnp.sqrt(jnp.float32(1.0) + EPS)
    s1 = (gamma1 * inv).reshape(1, D)
    s2 = (gamma2 * inv).reshape(1, D)
    b1r, t1 = b1.reshape(1, D), beta1.reshape(1, D)
    b2r, t2 = b2.reshape(1, D), beta2.reshape(1, D)

    h1 = _pack32(_matmul(x, W1p))
    p1 = _sc_scatter(h1, src, dst, eww, zeros)
    h2 = _pack32(_act_matmul(p1, b1r, s1, t1, W2p))
    p2 = _sc_scatter(h2, src, dst, eww, zeros)
    return _act_only(p2, b2r, s2, t2)
